# baseline (speedup 1.0000x reference)
import jax
import jax.numpy as jnp
from jax import lax
from jax.experimental import pallas as pl
from jax.experimental.pallas import tpu as pltpu
from jax.experimental.pallas import tpu_sc as plsc

_TOPK = 5
_B = 4
_V = 32
_PD = 7
_NC = 2
_NS = 16
_NEG = -1e30
_BIG = 1 << 30


def _topk_row(w0, w1, iota):
    idxs, vals = [], []
    for _ in range(_TOPK):
        m0 = jnp.max(w0)
        m1 = jnp.max(w1)
        use0 = m0 >= m1
        i0 = jnp.min(jnp.where(w0 == m0, iota, _BIG))
        i1 = jnp.min(jnp.where(w1 == m1, iota, _BIG))
        idxs.append(jnp.where(use0, i0, i1 + 16))
        vals.append(jnp.where(use0, m0, m1))
        w0 = jnp.where((iota == i0) & use0, _NEG, w0)
        w1 = jnp.where((iota == i1) & jnp.logical_not(use0), _NEG, w1)
    return idxs, vals


def _probs_vec(vals, iota):
    vals_v = jnp.full((16,), _NEG, jnp.float32)
    for t in range(_TOPK):
        vals_v = jnp.where(iota == t, vals[t], vals_v)
    e = jnp.exp(vals_v - vals[0])
    e = jnp.where(iota < _TOPK, e, 0.0)
    return e / jnp.sum(e)


def _body(sel_hbm, img_hbm, pose_hbm, out_img, out_pose, out_prob,
          sel_v, pose_v, pose_o, prob_o):
    wid = lax.axis_index("s") * _NC + lax.axis_index("c")
    iota = lax.iota(jnp.int32, 16)

    @pl.when(wid < _B * _TOPK)
    def _():
        pltpu.sync_copy(sel_hbm, sel_v)
        b = wid // _TOPK
        k = wid % _TOPK
        w0 = sel_v[pl.ds(b * _V, 16)]
        w1 = sel_v[pl.ds(b * _V + 16, 16)]
        idxs, _ = _topk_row(w0, w1, iota)
        idx_own = idxs[0]
        for t in range(1, _TOPK):
            idx_own = jnp.where(k == t, idxs[t], idx_own)
        pltpu.sync_copy(img_hbm.at[b, idx_own], out_img.at[b, k])

    @pl.when(wid == _B * _TOPK)
    def _():
        pltpu.sync_copy(sel_hbm, sel_v)
        pltpu.sync_copy(pose_hbm, pose_v.at[pl.ds(0, _B * _V * _PD)])
        for b in range(_B):
            w0 = sel_v[pl.ds(b * _V, 16)]
            w1 = sel_v[pl.ds(b * _V + 16, 16)]
            idxs, vals = _topk_row(w0, w1, iota)
            prob_o[pl.ds(b * _TOPK, 16)] = _probs_vec(vals, iota)
            for t in range(_TOPK):
                g = b * _V + idxs[t]
                row = pose_v[pl.ds(g * _PD, 16)]
                pose_o[pl.ds((b * _TOPK + t) * _PD, 16)] = row
        pltpu.sync_copy(pose_o.at[pl.ds(0, _B * _TOPK * _PD)], out_pose)
        pltpu.sync_copy(prob_o.at[pl.ds(0, _B * _TOPK)], out_prob)


_sc_call = pl.kernel(
    _body,
    out_type=(
        jax.ShapeDtypeStruct((_B, _TOPK, 128, 128, 3), jnp.float32),
        jax.ShapeDtypeStruct((_B * _TOPK * _PD,), jnp.float32),
        jax.ShapeDtypeStruct((_B * _TOPK,), jnp.float32),
    ),
    mesh=plsc.VectorSubcoreMesh(core_axis_name="c", subcore_axis_name="s"),
    scratch_types=[
        pltpu.VMEM((_B * _V,), jnp.float32),
        pltpu.VMEM((_B * _V * _PD + 16,), jnp.float32),
        pltpu.VMEM((_B * _TOPK * _PD + 16,), jnp.float32),
        pltpu.VMEM((_B * _TOPK + 16,), jnp.float32),
    ],
    compiler_params=pltpu.CompilerParams(
        needs_layout_passes=False, use_tc_tiling_on_sc=True),
)


@jax.jit
def kernel(selection_weights, images, poses):
    sel = selection_weights.reshape(_B * _V)
    pose = poses.reshape(_B * _V * _PD)
    out_img, out_pose, out_prob = _sc_call(sel, images, pose)
    return (
        out_img,
        out_pose.reshape(_B, _TOPK, _PD),
        out_prob.reshape(_B, _TOPK),
    )


# speedup vs baseline: 1.0001x; 1.0001x over previous
import jax
import jax.numpy as jnp
from jax import lax
from jax.experimental import pallas as pl
from jax.experimental.pallas import tpu as pltpu

_TOPK = 5
_B = 4
_V = 32
_PD = 7
_NEG = -1e30
_BIG = 1 << 30


def _body(sel_ref, pose_ref, img_hbm, out_pose_ref, out_prob_ref,
          out_img_hbm, sem):
    w = sel_ref[...]
    iotac = lax.broadcasted_iota(jnp.int32, (_B, _V), 1)

    idx_cols, val_cols = [], []
    for _ in range(_TOPK):
        m = jnp.max(w, axis=1, keepdims=True)
        eq = w == m
        idxc = jnp.min(jnp.where(eq, iotac, _BIG), axis=1, keepdims=True)
        idx_cols.append(idxc)
        val_cols.append(m)
        w = jnp.where(iotac == idxc, _NEG, w)

    vals = jnp.concatenate(val_cols, axis=1)
    e = jnp.exp(vals - val_cols[0])
    out_prob_ref[...] = e / jnp.sum(e, axis=1, keepdims=True)

    poses = pose_ref[...]
    for t in range(_TOPK):
        oh = (iotac == idx_cols[t]).astype(jnp.float32)
        out_pose_ref[:, t, :] = jnp.sum(oh[:, :, None] * poses, axis=1)

    copies = []
    for b in range(_B):
        for t in range(_TOPK):
            idx_s = idx_cols[t][b, 0]
            cp = pltpu.make_async_copy(
                img_hbm.at[b, idx_s], out_img_hbm.at[b, t], sem)
            cp.start()
            copies.append(cp)
    for cp in copies:
        cp.wait()


_grid_spec = pltpu.PrefetchScalarGridSpec(
    num_scalar_prefetch=0,
    grid=(),
    in_specs=[
        pl.BlockSpec(memory_space=pltpu.VMEM),
        pl.BlockSpec(memory_space=pltpu.VMEM),
        pl.BlockSpec(memory_space=pltpu.MemorySpace.HBM),
    ],
    out_specs=[
        pl.BlockSpec(memory_space=pltpu.VMEM),
        pl.BlockSpec(memory_space=pltpu.VMEM),
        pl.BlockSpec(memory_space=pltpu.MemorySpace.HBM),
    ],
    scratch_shapes=[pltpu.SemaphoreType.DMA],
)

_call = pl.pallas_call(
    _body,
    grid_spec=_grid_spec,
    out_shape=(
        jax.ShapeDtypeStruct((_B, _TOPK, _PD), jnp.float32),
        jax.ShapeDtypeStruct((_B, _TOPK), jnp.float32),
        jax.ShapeDtypeStruct((_B, _TOPK, 128, 128, 3), jnp.float32),
    ),
)


@jax.jit
def kernel(selection_weights, images, poses):
    out_pose, out_prob, out_img = _call(selection_weights, poses, images)
    return (out_img, out_pose, out_prob)


# speedup vs baseline: 7.9105x; 7.9096x over previous
import jax
import jax.numpy as jnp
from jax import lax
from jax.experimental import pallas as pl
from jax.experimental.pallas import tpu as pltpu

_TOPK = 5
_B = 4
_V = 32
_PD = 7
_NEG = -1e30
_BIG = 1 << 30


def _select_body(sel_ref, pose_ref, idx_ref, out_pose_ref, out_prob_ref):
    w = sel_ref[...]
    iotac = lax.broadcasted_iota(jnp.int32, (_B, _V), 1)

    idx_cols, val_cols = [], []
    for _ in range(_TOPK):
        m = jnp.max(w, axis=1, keepdims=True)
        eq = w == m
        idxc = jnp.min(jnp.where(eq, iotac, _BIG), axis=1, keepdims=True)
        idx_cols.append(idxc)
        val_cols.append(m)
        w = jnp.where(iotac == idxc, _NEG, w)

    idx_cols += [idx_cols[0]] * (8 - _TOPK)
    idx_ref[...] = jnp.concatenate(idx_cols, axis=1)

    vals = jnp.concatenate(val_cols, axis=1)
    e = jnp.exp(vals - val_cols[0])
    out_prob_ref[...] = e / jnp.sum(e, axis=1, keepdims=True)

    poses = pose_ref[...]
    for t in range(_TOPK):
        oh = (iotac == idx_cols[t]).astype(jnp.float32)
        out_pose_ref[:, t, :] = jnp.sum(oh[:, :, None] * poses, axis=1)


_select_call = pl.pallas_call(
    _select_body,
    out_shape=(
        jax.ShapeDtypeStruct((_B, 8), jnp.int32),
        jax.ShapeDtypeStruct((_B, _TOPK, _PD), jnp.float32),
        jax.ShapeDtypeStruct((_B, _TOPK), jnp.float32),
    ),
)


def _gather_body(idx_sref, img_ref, out_ref):
    out_ref[...] = img_ref[...]


_gather_call = pl.pallas_call(
    _gather_body,
    grid_spec=pltpu.PrefetchScalarGridSpec(
        num_scalar_prefetch=1,
        grid=(_B * _TOPK,),
        in_specs=[
            pl.BlockSpec(
                (1, 1, 128, 128, 3),
                lambda i, idx: (i // _TOPK, idx[i // _TOPK, i % _TOPK],
                                0, 0, 0)),
        ],
        out_specs=pl.BlockSpec(
            (1, 1, 128, 128, 3),
            lambda i, idx: (i // _TOPK, i % _TOPK, 0, 0, 0)),
    ),
    out_shape=jax.ShapeDtypeStruct((_B, _TOPK, 128, 128, 3), jnp.float32),
)


@jax.jit
def kernel(selection_weights, images, poses):
    idx, out_pose, out_prob = _select_call(selection_weights, poses)
    out_img = _gather_call(idx, images)
    return (out_img, out_pose, out_prob)


# speedup vs baseline: 68.4900x; 8.6581x over previous
import jax
import jax.numpy as jnp
from jax import lax
from jax.experimental import pallas as pl
from jax.experimental.pallas import tpu as pltpu

_TOPK = 5
_B = 4
_V = 32
_PD = 7
_NEG = -1e30
_BIG = 1 << 30


def _select_body(sel_ref, pose_ref, idx_ref, out_pose_ref, out_prob_ref):
    w = sel_ref[...]
    iotac = lax.broadcasted_iota(jnp.int32, (_B, _V), 1)

    idx_cols, val_cols = [], []
    for _ in range(_TOPK):
        m = jnp.max(w, axis=1, keepdims=True)
        eq = w == m
        idxc = jnp.min(jnp.where(eq, iotac, _BIG), axis=1, keepdims=True)
        idx_cols.append(idxc)
        val_cols.append(m)
        w = jnp.where(iotac == idxc, _NEG, w)

    idx_cols += [idx_cols[0]] * (8 - _TOPK)
    idx_ref[...] = jnp.concatenate(idx_cols, axis=1)

    vals = jnp.concatenate(val_cols, axis=1)
    e = jnp.exp(vals - val_cols[0])
    out_prob_ref[...] = e / jnp.sum(e, axis=1, keepdims=True)

    poses = pose_ref[...]
    for t in range(_TOPK):
        oh = (iotac == idx_cols[t]).astype(jnp.float32)
        out_pose_ref[:, t, :] = jnp.sum(oh[:, :, None] * poses, axis=1)


_select_call = pl.pallas_call(
    _select_body,
    out_shape=(
        jax.ShapeDtypeStruct((_B, 8), jnp.int32),
        jax.ShapeDtypeStruct((_B, _TOPK, _PD), jnp.float32),
        jax.ShapeDtypeStruct((_B, _TOPK), jnp.float32),
    ),
)


def _gather_body(idx_sref, img_ref, out_ref):
    out_ref[...] = img_ref[...]


_gather_call = pl.pallas_call(
    _gather_body,
    grid_spec=pltpu.PrefetchScalarGridSpec(
        num_scalar_prefetch=1,
        grid=(_B * _TOPK,),
        in_specs=[
            pl.BlockSpec(
                (1, 1, 128, 384),
                lambda i, idx: (i // _TOPK, idx[i // _TOPK, i % _TOPK],
                                0, 0)),
        ],
        out_specs=pl.BlockSpec(
            (1, 1, 128, 384),
            lambda i, idx: (i // _TOPK, i % _TOPK, 0, 0)),
    ),
    out_shape=jax.ShapeDtypeStruct((_B, _TOPK, 128, 384), jnp.float32),
)


@jax.jit
def kernel(selection_weights, images, poses):
    idx, out_pose, out_prob = _select_call(selection_weights, poses)
    out_img = _gather_call(idx, images.reshape(_B, _V, 128, 384))
    return (out_img.reshape(_B, _TOPK, 128, 128, 3), out_pose, out_prob)


# speedup vs baseline: 316.8705x; 4.6265x over previous
import jax
import jax.numpy as jnp
from jax import lax
from jax.experimental import pallas as pl
from jax.experimental.pallas import tpu as pltpu

_TOPK = 5
_B = 4
_V = 32
_PD = 7
_NEG = -1e30
_BIG = 1 << 30


def _select_body(sel_ref, pose_ref, idx_ref, out_pose_ref, out_prob_ref):
    w = sel_ref[...]
    iotac = lax.broadcasted_iota(jnp.int32, (_B, _V), 1)

    idx_cols, val_cols = [], []
    for _ in range(_TOPK):
        m = jnp.max(w, axis=1, keepdims=True)
        eq = w == m
        idxc = jnp.min(jnp.where(eq, iotac, _BIG), axis=1, keepdims=True)
        idx_cols.append(idxc)
        val_cols.append(m)
        w = jnp.where(iotac == idxc, _NEG, w)

    idx_ref[...] = jnp.concatenate(
        idx_cols + [idx_cols[0]] * (8 - _TOPK), axis=1)

    vals = jnp.concatenate(val_cols, axis=1)
    e = jnp.exp(vals - val_cols[0])
    out_prob_ref[...] = e / jnp.sum(e, axis=1, keepdims=True)

    poses_t = pose_ref[...]
    for t in range(_TOPK):
        oh = (iotac == idx_cols[t]).astype(jnp.float32)
        out_pose_ref[:, :, t] = jnp.sum(oh[None, :, :] * poses_t, axis=2)


_select_call = pl.pallas_call(
    _select_body,
    out_shape=(
        jax.ShapeDtypeStruct((_B, 8), jnp.int32),
        jax.ShapeDtypeStruct((_PD, _B, _TOPK), jnp.float32),
        jax.ShapeDtypeStruct((_B, _TOPK), jnp.float32),
    ),
)


def _gather_body(idx_sref, img_ref, out_ref):
    out_ref[...] = img_ref[...]


_gather_call = pl.pallas_call(
    _gather_body,
    grid_spec=pltpu.PrefetchScalarGridSpec(
        num_scalar_prefetch=1,
        grid=(_B * _TOPK,),
        in_specs=[
            pl.BlockSpec(
                (1, 1, 3, 128, 128),
                lambda i, idx: (i // _TOPK, idx[i // _TOPK, i % _TOPK],
                                0, 0, 0)),
        ],
        out_specs=pl.BlockSpec(
            (1, 1, 3, 128, 128),
            lambda i, idx: (i // _TOPK, i % _TOPK, 0, 0, 0)),
    ),
    out_shape=jax.ShapeDtypeStruct((_B, _TOPK, 3, 128, 128), jnp.float32),
)


@jax.jit
def kernel(selection_weights, images, poses):
    imgs_t = jnp.transpose(images, (0, 1, 4, 2, 3))
    poses_t = jnp.transpose(poses, (2, 0, 1))
    idx, out_pose_t, out_prob = _select_call(selection_weights, poses_t)
    out_img_t = _gather_call(idx, imgs_t)
    return (
        jnp.transpose(out_img_t, (0, 1, 3, 4, 2)),
        jnp.transpose(out_pose_t, (1, 2, 0)),
        out_prob,
    )
